# baseline (speedup 1.0000x reference)
import functools

import jax
import jax.numpy as jnp
from jax import lax
from jax.experimental import pallas as pl
from jax.experimental.pallas import tpu as pltpu
from jax.experimental.pallas import tpu_sc as plsc

NT = 32768
NE = 64
NC = 2
NS = 16
NW = NC * NS
TPW = NT // NW
CHUNK = 64
NPAIRS = TPW // (2 * CHUNK)
L = 16
NV = NE // L
CW = CHUNK * NE

_mesh = plsc.VectorSubcoreMesh(core_axis_name="c", subcore_axis_name="s")


@functools.partial(
    pl.kernel,
    out_type=[
        jax.ShapeDtypeStruct((NT,), jnp.int32),
        jax.ShapeDtypeStruct((NT,), jnp.float32),
        jax.ShapeDtypeStruct((NC * NE,), jnp.int32),
        jax.ShapeDtypeStruct((NC * NE,), jnp.float32),
    ],
    mesh=_mesh,
    compiler_params=pltpu.CompilerParams(needs_layout_passes=False),
    scratch_types=[
        pltpu.VMEM((TPW * NE,), jnp.float32),
        pltpu.VMEM((TPW,), jnp.int32),
        pltpu.VMEM((TPW,), jnp.float32),
        pltpu.VMEM((NE,), jnp.int32),
        pltpu.VMEM((NE,), jnp.float32),
        pltpu.VMEM((NS * NE,), jnp.float32),
        pltpu.VMEM((NS * NE,), jnp.int32),
        pltpu.VMEM_SHARED((NS * NE,), jnp.float32),
        pltpu.VMEM_SHARED((NS * NE,), jnp.int32),
        pltpu.SemaphoreType.DMA,
        pltpu.SemaphoreType.DMA,
    ],
)
def _gate_kernel(x_hbm, sample_hbm, mult_hbm, cnt_hbm, psum_hbm,
                 big, sample_buf, mult_buf, cnt_buf, psum_buf,
                 agg_ps, agg_ct, sh_ps, sh_ct, sem0, sem1):
    cid = lax.axis_index("c")
    sid = lax.axis_index("s")
    wid = cid * NS + sid
    tok0 = wid * TPW
    word0 = tok0 * NE
    last_off = word0 + TPW * NE - CW

    idx0 = lax.iota(jnp.int32, L)
    ones_i = jnp.ones((L,), jnp.int32)
    z16f = jnp.zeros((L,), jnp.float32)
    z16i = jnp.zeros((L,), jnp.int32)
    idxc = [idx0 + j * L for j in range(NV)]
    lmask = [idx0 == t for t in range(L)]
    big_i = jnp.full((L,), NE, jnp.int32)
    perms = [jnp.bitwise_xor(idx0, 1 << k) for k in range(4)]

    _dnums = lax.GatherDimensionNumbers(
        offset_dims=(), collapsed_slice_dims=(0,), start_index_map=(0,))

    def _take(v, pm):
        return lax.gather(v, pm[:, None], _dnums, (1,),
                          mode=lax.GatherScatterMode.PROMISE_IN_BOUNDS)

    def _bfly(v, op):
        for pm in perms:
            v = op(v, _take(v, pm))
        return v

    def process(out_off, ps):
        ps = list(ps)
        for blk in range(CHUNK // L):
            svec = z16i
            mvec = z16f
            for tt in range(L):
                t = blk * L + tt
                l = [big[pl.ds((out_off + t) * NE + j * L, L)]
                     for j in range(NV)]
                m = _bfly(jnp.maximum(jnp.maximum(l[0], l[1]),
                                      jnp.maximum(l[2], l[3])), jnp.maximum)
                ex = [jnp.exp(l[j] - m) for j in range(NV)]
                s = _bfly((ex[0] + ex[1]) + (ex[2] + ex[3]), jnp.add)
                r = 1.0 / s
                c = [jnp.where(l[j] == m, idxc[j], big_i) for j in range(NV)]
                samp = _bfly(jnp.minimum(jnp.minimum(c[0], c[1]),
                                         jnp.minimum(c[2], c[3])),
                             jnp.minimum)
                for j in range(NV):
                    ps[j] = ps[j] + ex[j] * r
                svec = jnp.where(lmask[tt], samp, svec)
                mvec = jnp.where(lmask[tt], r, mvec)
            sample_buf[pl.ds(out_off + blk * L, L)] = svec
            mult_buf[pl.ds(out_off + blk * L, L)] = mvec
            plsc.addupdate_scatter(cnt_buf, [svec], ones_i)
        return ps

    for j in range(NV):
        cnt_buf[pl.ds(j * L, L)] = z16i

    HW = TPW * NE // 2
    HT = TPW // 2
    cp0 = pltpu.async_copy(x_hbm.at[pl.ds(word0, HW)],
                           big.at[pl.ds(0, HW)], sem0)
    cp1 = pltpu.async_copy(x_hbm.at[pl.ds(word0 + HW, HW)],
                           big.at[pl.ds(HW, HW)], sem1)

    ps = (z16f,) * NV
    for h, cp in ((0, cp0), (1, cp1)):
        cp.wait()

        def chunk_body(ci, ps, h=h):
            return tuple(process(h * HT + ci * CHUNK, list(ps)))

        ps = lax.fori_loop(0, HT // CHUNK, chunk_body, ps)

    for j in range(NV):
        psum_buf[pl.ds(j * L, L)] = ps[j]

    pltpu.sync_copy(sample_buf, sample_hbm.at[pl.ds(tok0, TPW)])
    pltpu.sync_copy(mult_buf, mult_hbm.at[pl.ds(tok0, TPW)])

    pltpu.sync_copy(psum_buf, sh_ps.at[pl.ds(sid * NE, NE)])
    pltpu.sync_copy(cnt_buf, sh_ct.at[pl.ds(sid * NE, NE)])
    plsc.subcore_barrier()

    @pl.when(sid == 0)
    def _():
        pltpu.sync_copy(sh_ps, agg_ps)
        pltpu.sync_copy(sh_ct, agg_ct)
        accp = [z16f for _ in range(NV)]
        accc = [z16i for _ in range(NV)]
        for rr in range(NS):
            for j in range(NV):
                accp[j] = accp[j] + agg_ps[pl.ds(rr * NE + j * L, L)]
                accc[j] = accc[j] + agg_ct[pl.ds(rr * NE + j * L, L)]
        for j in range(NV):
            psum_buf[pl.ds(j * L, L)] = accp[j]
            cnt_buf[pl.ds(j * L, L)] = accc[j]
        pltpu.sync_copy(psum_buf, psum_hbm.at[pl.ds(cid * NE, NE)])
        pltpu.sync_copy(cnt_buf, cnt_hbm.at[pl.ds(cid * NE, NE)])


def _loss_body(cnt_ref, ps_ref, out_ref):
    cntf = cnt_ref[...].astype(jnp.float32)
    ps = ps_ref[...]
    f2 = jnp.sum(cntf, axis=0, keepdims=True) * (1.0 / NT)
    pm2 = jnp.sum(ps, axis=0, keepdims=True) * (1.0 / NT)
    out_ref[...] = jnp.float32(NE) * jnp.sum(pm2 * f2, axis=1, keepdims=True)


def kernel(logits):
    x = logits.reshape(-1)
    sample, mult, cnt, psum = _gate_kernel(x)
    loss = pl.pallas_call(
        _loss_body,
        out_shape=jax.ShapeDtypeStruct((1, 1), jnp.float32),
    )(cnt.reshape(NC, NE), psum.reshape(NC, NE))
    return sample, mult.reshape(NT, 1), loss.reshape(())


# speedup vs baseline: 1.3854x; 1.3854x over previous
import functools

import jax
import jax.numpy as jnp
from jax import lax
from jax.experimental import pallas as pl
from jax.experimental.pallas import tpu as pltpu
from jax.experimental.pallas import tpu_sc as plsc

NT = 32768
NE = 64
NC = 2
NS = 16
NW = NC * NS
TPW = NT // NW
CHUNK = 64
NPAIRS = TPW // (2 * CHUNK)
L = 16
NV = NE // L
CW = CHUNK * NE

_mesh = plsc.VectorSubcoreMesh(core_axis_name="c", subcore_axis_name="s")


@functools.partial(
    pl.kernel,
    out_type=[
        jax.ShapeDtypeStruct((NT,), jnp.int32),
        jax.ShapeDtypeStruct((NT,), jnp.float32),
        jax.ShapeDtypeStruct((NC * NE,), jnp.int32),
        jax.ShapeDtypeStruct((NC * NE,), jnp.float32),
    ],
    mesh=_mesh,
    compiler_params=pltpu.CompilerParams(needs_layout_passes=False),
    scratch_types=[
        pltpu.VMEM((4 * CHUNK, NE), jnp.float32),
        pltpu.VMEM((4 * CHUNK, NE), jnp.float32),
        pltpu.VMEM((TPW,), jnp.int32),
        pltpu.VMEM((TPW,), jnp.float32),
        pltpu.VMEM((NE,), jnp.int32),
        pltpu.VMEM((NE,), jnp.float32),
        pltpu.VMEM((NS * NE,), jnp.float32),
        pltpu.VMEM((NS * NE,), jnp.int32),
        pltpu.VMEM_SHARED((NS * NE,), jnp.float32),
        pltpu.VMEM_SHARED((NS * NE,), jnp.int32),
        pltpu.SemaphoreType.DMA,
        pltpu.SemaphoreType.DMA,
    ],
)
def _gate_kernel(x_hbm, sample_hbm, mult_hbm, cnt_hbm, psum_hbm,
                 bufA, bufB, sample_buf, mult_buf, cnt_buf, psum_buf,
                 agg_ps, agg_ct, sh_ps, sh_ct, sem0, sem1):
    cid = lax.axis_index("c")
    sid = lax.axis_index("s")
    wid = cid * NS + sid
    tok0 = wid * TPW
    word0 = tok0 * NE
    last_off = word0 + TPW * NE - CW

    idx0 = lax.iota(jnp.int32, L)
    ones_i = jnp.ones((L,), jnp.int32)
    z16f = jnp.zeros((L,), jnp.float32)
    z16i = jnp.zeros((L,), jnp.int32)
    idxc = [idx0 + j * L for j in range(NV)]
    lmask = [idx0 == t for t in range(L)]
    big_i = jnp.full((L,), NE, jnp.int32)
    perms = [jnp.bitwise_xor(idx0, 1 << k) for k in range(4)]

    _dnums = lax.GatherDimensionNumbers(
        offset_dims=(), collapsed_slice_dims=(0,), start_index_map=(0,))

    def _take(v, pm):
        return lax.gather(v, pm[:, None], _dnums, (1,),
                          mode=lax.GatherScatterMode.PROMISE_IN_BOUNDS)

    def _bfly(v, op):
        for pm in perms:
            v = op(v, _take(v, pm))
        return v

    def process(buf, in_off, out_off, ps):
        ps = list(ps)
        for blk in range(CHUNK // L):
            svec = z16i
            mvec = z16f
            for tt in range(L):
                t = blk * L + tt
                l = [buf[in_off + t, pl.ds(j * L, L)]
                     for j in range(NV)]
                m = jnp.max(jnp.maximum(jnp.maximum(l[0], l[1]),
                                        jnp.maximum(l[2], l[3])))
                ex = [jnp.exp(l[j] - m) for j in range(NV)]
                s = jnp.sum((ex[0] + ex[1]) + (ex[2] + ex[3]))
                r = 1.0 / jnp.broadcast_to(s, (L,))
                c = [jnp.where(l[j] == m, idxc[j], big_i) for j in range(NV)]
                samp = jnp.min(jnp.minimum(jnp.minimum(c[0], c[1]),
                                           jnp.minimum(c[2], c[3])))
                for j in range(NV):
                    ps[j] = ps[j] + ex[j] * r
                svec = jnp.where(lmask[tt], samp, svec)
                mvec = jnp.where(lmask[tt], r, mvec)
            sample_buf[pl.ds(out_off + blk * L, L)] = svec
            mult_buf[pl.ds(out_off + blk * L, L)] = mvec
            plsc.addupdate_scatter(cnt_buf, [svec], ones_i)
        return ps

    for j in range(NV):
        cnt_buf[pl.ds(j * L, L)] = z16i

    SCT = 4 * CHUNK
    NSC = TPW // SCT

    def start_copy(si, buf, sem):
        return pltpu.async_copy(x_hbm.at[pl.ds(tok0 + si * SCT, SCT), :],
                                buf.at[pl.ds(0, SCT), :], sem)

    bufs = (bufA, bufB)
    sems = (sem0, sem1)
    cps = [start_copy(0, bufA, sem0), start_copy(1, bufB, sem1)]

    ps = (z16f,) * NV
    for si in range(NSC):
        b = si % 2
        cps[si].wait()

        def chunk_body(ci, ps, b=b, si=si):
            return tuple(process(bufs[b], ci * CHUNK,
                                 si * SCT + ci * CHUNK, list(ps)))

        ps = lax.fori_loop(0, SCT // CHUNK, chunk_body, ps)
        if si + 2 < NSC:
            cps.append(start_copy(si + 2, bufs[b], sems[b]))

    for j in range(NV):
        psum_buf[pl.ds(j * L, L)] = ps[j]

    pltpu.sync_copy(sample_buf, sample_hbm.at[pl.ds(tok0, TPW)])
    pltpu.sync_copy(mult_buf, mult_hbm.at[pl.ds(tok0, TPW)])

    pltpu.sync_copy(psum_buf, sh_ps.at[pl.ds(sid * NE, NE)])
    pltpu.sync_copy(cnt_buf, sh_ct.at[pl.ds(sid * NE, NE)])
    plsc.subcore_barrier()

    @pl.when(sid == 0)
    def _():
        pltpu.sync_copy(sh_ps, agg_ps)
        pltpu.sync_copy(sh_ct, agg_ct)
        accp = [z16f for _ in range(NV)]
        accc = [z16i for _ in range(NV)]
        for rr in range(NS):
            for j in range(NV):
                accp[j] = accp[j] + agg_ps[pl.ds(rr * NE + j * L, L)]
                accc[j] = accc[j] + agg_ct[pl.ds(rr * NE + j * L, L)]
        for j in range(NV):
            psum_buf[pl.ds(j * L, L)] = accp[j]
            cnt_buf[pl.ds(j * L, L)] = accc[j]
        pltpu.sync_copy(psum_buf, psum_hbm.at[pl.ds(cid * NE, NE)])
        pltpu.sync_copy(cnt_buf, cnt_hbm.at[pl.ds(cid * NE, NE)])


def _loss_body(cnt_ref, ps_ref, out_ref):
    cntf = cnt_ref[...].astype(jnp.float32)
    ps = ps_ref[...]
    f2 = (cntf[0:1, :NE] + cntf[0:1, NE:]) * (1.0 / NT)
    pm2 = (ps[0:1, :NE] + ps[0:1, NE:]) * (1.0 / NT)
    out_ref[...] = jnp.float32(NE) * jnp.sum(pm2 * f2, axis=1, keepdims=True)


def kernel(logits):
    sample, mult, cnt, psum = _gate_kernel(logits)
    loss = pl.pallas_call(
        _loss_body,
        out_shape=jax.ShapeDtypeStruct((1, 1), jnp.float32),
    )(cnt.reshape(1, NC * NE), psum.reshape(1, NC * NE))
    return sample, mult.reshape(NT, 1), loss.reshape(())
